# Initial kernel scaffold; baseline (speedup 1.0000x reference)
import jax
import jax.numpy as jnp
from jax.experimental import pallas as pl


def kernel(x, edge_index, W_l, b_l, W_r, bn_gamma, bn_beta, W1, b1, W2, b2, W3, b3):
    raise NotImplementedError("write your pallas kernel here")


# speedup vs baseline: 8.5078x; 8.5078x over previous
import functools

import jax
import jax.numpy as jnp
from jax import lax
from jax.experimental import pallas as pl
from jax.experimental.pallas import tpu as pltpu
from jax.experimental.pallas import tpu_sc as plsc

N = 10000
E = 320000
D = 128

_NC = 2
_NS = 16
_NW = _NC * _NS
_K = 200
_EPW = E // _NW
_CHUNKS = _EPW // _K
_RPS = 624
_TAIL = N - _NS * _RPS


def _make_agg():
    mesh = plsc.VectorSubcoreMesh(core_axis_name="c", subcore_axis_name="s")

    @functools.partial(
        pl.kernel,
        mesh=mesh,
        out_type=jax.ShapeDtypeStruct((_NC, N, D), jnp.float32),
        scratch_types=[
            pltpu.VMEM((_K,), jnp.int32),
            pltpu.VMEM((_K,), jnp.int32),
            pltpu.VMEM((_K, D), jnp.float32),
            pltpu.VMEM_SHARED((N, D), jnp.float32),
            pltpu.SemaphoreType.DMA,
        ],
    )
    def agg_kernel(x_hbm, edge_hbm, out_hbm, src_v, dst_v, rows_v, agg_sh, sem):
        c = lax.axis_index("c")
        s = lax.axis_index("s")

        def zero_row(i, carry):
            for j in range(D // 16):
                rows_v[i, pl.ds(j * 16, 16)] = jnp.zeros((16,), jnp.float32)
            return carry

        lax.fori_loop(0, _K, zero_row, 0)
        r0 = s * _RPS
        for off in range(0, _RPS, _K):
            w = min(_K, _RPS - off)
            pltpu.sync_copy(rows_v.at[pl.ds(0, w)], agg_sh.at[pl.ds(r0 + off, w)])

        @pl.when(s == _NS - 1)
        def _zero_tail():
            pltpu.sync_copy(rows_v.at[pl.ds(0, _TAIL)],
                            agg_sh.at[pl.ds(_NS * _RPS, _TAIL)])

        plsc.subcore_barrier()

        e0 = (c * _NS + s) * _EPW

        def body(i, carry):
            base = e0 + i * _K
            pltpu.sync_copy(edge_hbm.at[pl.ds(base, _K)], src_v)
            pltpu.sync_copy(edge_hbm.at[pl.ds(E + base, _K)], dst_v)
            pltpu.async_copy(x_hbm.at[src_v], rows_v, sem).wait()
            pltpu.sync_copy(rows_v, agg_sh.at[dst_v], add=True)
            return carry

        lax.fori_loop(0, _CHUNKS, body, 0)

        plsc.subcore_barrier()
        pltpu.sync_copy(agg_sh.at[pl.ds(r0, _RPS)],
                        out_hbm.at[c, pl.ds(r0, _RPS)])

        @pl.when(s == _NS - 1)
        def _write_tail():
            pltpu.sync_copy(agg_sh.at[pl.ds(_NS * _RPS, _TAIL)],
                            out_hbm.at[c, pl.ds(_NS * _RPS, _TAIL)])

    return agg_kernel


_AGG = _make_agg()

_R = 2000


def _dense_body(p_ref, x_ref, wl_ref, wr_ref, w1_ref, w2_ref, w3_ref,
                b0_ref, b1_ref, b2_ref, b3_ref, out_ref):
    agg = p_ref[0] + p_ref[1]
    h = (jnp.dot(agg, wl_ref[...], preferred_element_type=jnp.float32)
         + jnp.dot(x_ref[...], wr_ref[...], preferred_element_type=jnp.float32)
         + b0_ref[...])
    h = jnp.maximum(h, 0.0)
    h = jnp.maximum(
        jnp.dot(h, w1_ref[...], preferred_element_type=jnp.float32) + b1_ref[...], 0.0)
    h = jnp.maximum(
        jnp.dot(h, w2_ref[...], preferred_element_type=jnp.float32) + b2_ref[...], 0.0)
    out_ref[...] = (
        jnp.dot(h, w3_ref[...], preferred_element_type=jnp.float32) + b3_ref[...])


def _dense(p, x, wl_t, wr_t, w1_t, w2_t, w3_t, b0, b1, b2, b3):
    full = lambda i: (0, 0)
    return pl.pallas_call(
        _dense_body,
        grid=(N // _R,),
        in_specs=[
            pl.BlockSpec((_NC, _R, D), lambda i: (0, i, 0)),
            pl.BlockSpec((_R, D), lambda i: (i, 0)),
            pl.BlockSpec((D, D), full),
            pl.BlockSpec((D, D), full),
            pl.BlockSpec((D, D), full),
            pl.BlockSpec((D, D), full),
            pl.BlockSpec((D, D), full),
            pl.BlockSpec((1, D), full),
            pl.BlockSpec((1, D), full),
            pl.BlockSpec((1, D), full),
            pl.BlockSpec((1, D), full),
        ],
        out_specs=pl.BlockSpec((_R, D), lambda i: (i, 0)),
        out_shape=jax.ShapeDtypeStruct((N, D), jnp.float32),
    )(p, x, wl_t, wr_t, w1_t, w2_t, w3_t, b0, b1, b2, b3)


def kernel(x, edge_index, W_l, b_l, W_r, bn_gamma, bn_beta, W1, b1, W2, b2, W3, b3):
    sc = bn_gamma / jnp.sqrt(1.0 + 1e-5)
    wl_t = W_l.T * sc[None, :]
    wr_t = W_r.T * sc[None, :]
    b0 = (b_l * sc + bn_beta)[None, :]
    partials = _AGG(x, edge_index.reshape(2 * E))
    return _dense(partials, x, wl_t, wr_t, W1.T, W2.T, W3.T,
                  b0, b1[None, :], b2[None, :], b3[None, :])
